# baseline (speedup 1.0000x reference)
import functools

import jax
import jax.numpy as jnp
from jax import lax
from jax.experimental import pallas as pl
from jax.experimental.pallas import tpu as pltpu
from jax.experimental.pallas import tpu_sc as plsc

N = 10000
E = 320000
D = 128

NC = 2
NS = 16
NW = NC * NS
CH = 128
NP = 10240
ROWS_PER_TILE = NP // NS
CPW = 80
EPAD = NW * CPW * CH
NBUF = 2


def _sc_agg_body(with_deg, *refs):
    if with_deg:
        (x_hbm, e4_hbm, zc_hbm, acc_out, deg_out,
         gsems, isems, acc_sh, degv) = refs[:9]
        bufs = refs[9:]
    else:
        (x_hbm, e4_hbm, zc_hbm, acc_out,
         gsems, isems, acc_sh) = refs[:7]
        bufs = refs[7:]
    rows = bufs[:NBUF]
    idxs = bufs[NBUF:2 * NBUF]

    c = lax.axis_index("c")
    s = lax.axis_index("s")
    w = s * NC + c
    rbase = s * ROWS_PER_TILE
    zeros16 = jnp.zeros((16,), jnp.float32)
    ones16 = jnp.ones((16,), jnp.float32)

    def idx_start(g, b):
        pltpu.async_copy(e4_hbm.at[w, g], idxs[b], isems.at[b])

    def idx_wait(g, b):
        pltpu.make_async_copy(e4_hbm.at[w, g], idxs[b], isems.at[b]).wait()

    def gather_start(b):
        pltpu.async_copy(x_hbm.at[idxs[b].at[0]], rows[b], gsems.at[b])

    def gather_wait(b):
        pltpu.make_async_copy(x_hbm.at[idxs[b].at[0]], rows[b],
                              gsems.at[b]).wait()

    pltpu.sync_copy(zc_hbm, acc_sh.at[pl.ds(rbase, ROWS_PER_TILE)])
    if with_deg:
        def zbody(i, carry):
            for k in range(16):
                degv[pl.ds((i * 16 + k) * 16, 16)] = zeros16
            return carry
        lax.fori_loop(0, NP // 256, zbody, 0)
    plsc.subcore_barrier()

    for b in range(NBUF - 1):
        idx_start(b, b)
        idx_wait(b, b)
        gather_start(b)
    idx_start(NBUF - 1, NBUF - 1)

    def outer(go, carry):
        for b in range(NBUF):
            g = go * NBUF + b
            bprev = (b + NBUF - 1) % NBUF
            g2 = g + NBUF - 1

            @pl.when(g2 < CPW)
            def _():
                idx_wait(g2, bprev)
                gather_start(bprev)

            gather_wait(b)
            pltpu.sync_copy(rows[b], acc_sh.at[idxs[b].at[1]], add=True)
            if with_deg:
                for j in range(CH // 16):
                    plsc.addupdate_scatter(
                        degv, [idxs[b][1, pl.ds(j * 16, 16)]], ones16)

            @pl.when(g + NBUF < CPW)
            def _():
                idx_start(g + NBUF, b)
        return carry

    lax.fori_loop(0, CPW // NBUF, outer, 0)
    plsc.subcore_barrier()

    pltpu.sync_copy(acc_sh.at[pl.ds(rbase, ROWS_PER_TILE)],
                    acc_out.at[c, pl.ds(rbase, ROWS_PER_TILE)])
    if with_deg:
        pltpu.sync_copy(degv, deg_out.at[w])


def _sc_aggregate(x, e4, with_deg):
    mesh = plsc.VectorSubcoreMesh(core_axis_name="c", subcore_axis_name="s")
    zc = jnp.zeros((ROWS_PER_TILE, D), jnp.float32)
    scratch = [
        pltpu.SemaphoreType.DMA((NBUF,)),
        pltpu.SemaphoreType.DMA((NBUF,)),
        pltpu.VMEM_SHARED((NP, D), jnp.float32),
    ]
    if with_deg:
        scratch.append(pltpu.VMEM((NP,), jnp.float32))
        out_type = (jax.ShapeDtypeStruct((NC, NP, D), jnp.float32),
                    jax.ShapeDtypeStruct((NW, NP), jnp.float32))
    else:
        out_type = jax.ShapeDtypeStruct((NC, NP, D), jnp.float32)
    scratch += [pltpu.VMEM((CH, D), jnp.float32) for _ in range(NBUF)]
    scratch += [pltpu.VMEM((2, CH), jnp.int32) for _ in range(NBUF)]
    kern = pl.kernel(
        functools.partial(_sc_agg_body, with_deg),
        out_type=out_type,
        mesh=mesh,
        scratch_types=scratch,
        compiler_params=pltpu.CompilerParams(needs_layout_passes=False),
    )
    return kern(x, e4, zc)


def _tc_layer_body(relu, acc_ref, deg_ref, x_ref, wl_ref, wc_ref, b_ref, o_ref):
    a = acc_ref[0] + acc_ref[1]
    d = jnp.sum(deg_ref[...], axis=0)
    dclip = jnp.maximum(d, 1.0)[:, None]
    mean = a / dclip
    y = (jnp.dot(mean, wl_ref[...], preferred_element_type=jnp.float32)
         + jnp.dot(x_ref[...], wc_ref[...], preferred_element_type=jnp.float32)
         + b_ref[...])
    if relu:
        y = jnp.maximum(y, 0.0)
    o_ref[...] = y


def _tc_layer(acc, deg, x, wl, wc, b, relu):
    R = 2048
    grid = (NP // R,)
    return pl.pallas_call(
        functools.partial(_tc_layer_body, relu),
        grid=grid,
        in_specs=[
            pl.BlockSpec((NC, R, D), lambda i: (0, i, 0)),
            pl.BlockSpec((NW, R), lambda i: (0, i)),
            pl.BlockSpec((R, D), lambda i: (i, 0)),
            pl.BlockSpec((D, D), lambda i: (0, 0)),
            pl.BlockSpec((D, D), lambda i: (0, 0)),
            pl.BlockSpec((1, D), lambda i: (0, 0)),
        ],
        out_specs=pl.BlockSpec((R, D), lambda i: (i, 0)),
        out_shape=jax.ShapeDtypeStruct((NP, D), jnp.float32),
    )(acc, deg, x, wl, wc, b)


def kernel(x, edge_index, W1l, b1l, W1r, Wlin1, blin1, W2l, b2l, W2r, Wlin2, blin2):
    src = edge_index[0]
    dst = edge_index[1]
    src3 = jnp.concatenate(
        [src, jnp.zeros((EPAD - E,), jnp.int32)]).reshape(NW, CPW, 1, CH)
    dst3 = jnp.concatenate(
        [dst, jnp.full((EPAD - E,), NP - 1, jnp.int32)]).reshape(NW, CPW, 1, CH)
    e4 = jnp.concatenate([src3, dst3], axis=2)
    xp = jnp.concatenate([x, jnp.zeros((NP - N, D), jnp.float32)], axis=0)
    acc1, deg = _sc_aggregate(xp, e4, with_deg=True)
    h = _tc_layer(acc1, deg, xp, W1l, W1r + Wlin1,
                  (b1l + blin1).reshape(1, D), relu=True)
    acc2 = _sc_aggregate(h, e4, with_deg=False)
    out = _tc_layer(acc2, deg, h, W2l, W2r + Wlin2,
                    (b2l + blin2).reshape(1, D), relu=False)
    return out[:N]


# speedup vs baseline: 1.0049x; 1.0049x over previous
import functools

import jax
import jax.numpy as jnp
from jax import lax
from jax.experimental import pallas as pl
from jax.experimental.pallas import tpu as pltpu
from jax.experimental.pallas import tpu_sc as plsc

N = 10000
E = 320000
D = 128

NC = 2
NS = 16
NW = NC * NS
CH = 128
NP = 10240
ROWS_PER_TILE = NP // NS
CPW = 80
EPAD = NW * CPW * CH
NBUF = 2


def _sc_agg_body(with_deg, *refs):
    if with_deg:
        (x_hbm, src_hbm, dst_hbm, zc_hbm, acc_out, deg_out,
         gsems, isems, acc_sh, degv) = refs[:10]
        bufs = refs[10:]
    else:
        (x_hbm, src_hbm, dst_hbm, zc_hbm, acc_out,
         gsems, isems, acc_sh) = refs[:8]
        bufs = refs[8:]
    rows = bufs[:NBUF]
    sidxs = bufs[NBUF:2 * NBUF]
    didxs = bufs[2 * NBUF:3 * NBUF]

    c = lax.axis_index("c")
    s = lax.axis_index("s")
    w = s * NC + c
    rbase = s * ROWS_PER_TILE
    ebase = w * (CPW * CH)
    zeros16 = jnp.zeros((16,), jnp.float32)
    ones16 = jnp.ones((16,), jnp.float32)

    def idx_start(g, b):
        eb = ebase + g * CH
        pltpu.async_copy(src_hbm.at[pl.ds(eb, CH)], sidxs[b], isems.at[b])
        pltpu.async_copy(dst_hbm.at[pl.ds(eb, CH)], didxs[b], isems.at[b])

    def idx_wait(g, b):
        eb = ebase + g * CH
        pltpu.make_async_copy(src_hbm.at[pl.ds(eb, CH)], sidxs[b],
                              isems.at[b]).wait()
        pltpu.make_async_copy(dst_hbm.at[pl.ds(eb, CH)], didxs[b],
                              isems.at[b]).wait()

    def gather_start(b):
        pltpu.async_copy(x_hbm.at[sidxs[b]], rows[b], gsems.at[b])

    def gather_wait(b):
        pltpu.make_async_copy(x_hbm.at[sidxs[b]], rows[b],
                              gsems.at[b]).wait()

    pltpu.sync_copy(zc_hbm, acc_sh.at[pl.ds(rbase, ROWS_PER_TILE)])
    if with_deg:
        def zbody(i, carry):
            for k in range(16):
                degv[pl.ds((i * 16 + k) * 16, 16)] = zeros16
            return carry
        lax.fori_loop(0, NP // 256, zbody, 0)
    plsc.subcore_barrier()

    for b in range(NBUF - 1):
        idx_start(b, b)
        idx_wait(b, b)
        gather_start(b)
    idx_start(NBUF - 1, NBUF - 1)

    def outer(go, carry):
        for b in range(NBUF):
            g = go * NBUF + b
            bprev = (b + NBUF - 1) % NBUF
            g2 = g + NBUF - 1

            @pl.when(g2 < CPW)
            def _():
                idx_wait(g2, bprev)
                gather_start(bprev)

            gather_wait(b)
            pltpu.sync_copy(rows[b], acc_sh.at[didxs[b]], add=True)
            if with_deg:
                for j in range(CH // 16):
                    plsc.addupdate_scatter(
                        degv, [didxs[b][pl.ds(j * 16, 16)]], ones16)

            @pl.when(g + NBUF < CPW)
            def _():
                idx_start(g + NBUF, b)
        return carry

    lax.fori_loop(0, CPW // NBUF, outer, 0)
    plsc.subcore_barrier()

    pltpu.sync_copy(acc_sh.at[pl.ds(rbase, ROWS_PER_TILE)],
                    acc_out.at[c, pl.ds(rbase, ROWS_PER_TILE)])
    if with_deg:
        pltpu.sync_copy(degv, deg_out.at[w])


def _sc_aggregate(x, srcp, dstp, with_deg):
    mesh = plsc.VectorSubcoreMesh(core_axis_name="c", subcore_axis_name="s")
    zc = jnp.zeros((ROWS_PER_TILE, D), jnp.float32)
    scratch = [
        pltpu.SemaphoreType.DMA((NBUF,)),
        pltpu.SemaphoreType.DMA((NBUF,)),
        pltpu.VMEM_SHARED((NP, D), jnp.float32),
    ]
    if with_deg:
        scratch.append(pltpu.VMEM((NP,), jnp.float32))
        out_type = (jax.ShapeDtypeStruct((NC, NP, D), jnp.float32),
                    jax.ShapeDtypeStruct((NW, NP), jnp.float32))
    else:
        out_type = jax.ShapeDtypeStruct((NC, NP, D), jnp.float32)
    scratch += [pltpu.VMEM((CH, D), jnp.float32) for _ in range(NBUF)]
    scratch += [pltpu.VMEM((CH,), jnp.int32) for _ in range(2 * NBUF)]
    kern = pl.kernel(
        functools.partial(_sc_agg_body, with_deg),
        out_type=out_type,
        mesh=mesh,
        scratch_types=scratch,
        compiler_params=pltpu.CompilerParams(needs_layout_passes=False),
    )
    return kern(x, srcp, dstp, zc)


def _tc_layer_body(relu, acc_ref, deg_ref, x_ref, wl_ref, wc_ref, b_ref, o_ref):
    a = acc_ref[0] + acc_ref[1]
    d = jnp.sum(deg_ref[...], axis=0)
    dclip = jnp.maximum(d, 1.0)[:, None]
    mean = a / dclip
    y = (jnp.dot(mean, wl_ref[...], preferred_element_type=jnp.float32)
         + jnp.dot(x_ref[...], wc_ref[...], preferred_element_type=jnp.float32)
         + b_ref[...])
    if relu:
        y = jnp.maximum(y, 0.0)
    o_ref[...] = y


def _tc_layer(acc, deg, x, wl, wc, b, relu):
    R = 2048
    grid = (NP // R,)
    return pl.pallas_call(
        functools.partial(_tc_layer_body, relu),
        grid=grid,
        in_specs=[
            pl.BlockSpec((NC, R, D), lambda i: (0, i, 0)),
            pl.BlockSpec((NW, R), lambda i: (0, i)),
            pl.BlockSpec((R, D), lambda i: (i, 0)),
            pl.BlockSpec((D, D), lambda i: (0, 0)),
            pl.BlockSpec((D, D), lambda i: (0, 0)),
            pl.BlockSpec((1, D), lambda i: (0, 0)),
        ],
        out_specs=pl.BlockSpec((R, D), lambda i: (i, 0)),
        out_shape=jax.ShapeDtypeStruct((NP, D), jnp.float32),
    )(acc, deg, x, wl, wc, b)


def kernel(x, edge_index, W1l, b1l, W1r, Wlin1, blin1, W2l, b2l, W2r, Wlin2, blin2):
    src = edge_index[0]
    dst = edge_index[1]
    srcp = jnp.concatenate([src, jnp.zeros((EPAD - E,), jnp.int32)])
    dstp = jnp.concatenate([dst, jnp.full((EPAD - E,), NP - 1, jnp.int32)])
    xp = jnp.concatenate([x, jnp.zeros((NP - N, D), jnp.float32)], axis=0)
    acc1, deg = _sc_aggregate(xp, srcp, dstp, with_deg=True)
    h = _tc_layer(acc1, deg, xp, W1l, W1r + Wlin1,
                  (b1l + blin1).reshape(1, D), relu=True)
    acc2 = _sc_aggregate(h, srcp, dstp, with_deg=False)
    out = _tc_layer(acc2, deg, h, W2l, W2r + Wlin2,
                    (b2l + blin2).reshape(1, D), relu=False)
    return out[:N]
